# baseline (speedup 1.0000x reference)
import functools

import jax
import jax.numpy as jnp
from jax import lax
from jax.experimental import pallas as pl
from jax.experimental.pallas import tpu as pltpu
from jax.experimental.pallas import tpu_sc as plsc

D = 128
D2 = 2 * D
HID = 2 * D2
H4 = 4 * HID
B = 1024
FEW = 5
STEPS = 4


_NW = 32
_RPW = B // _NW
_CH = 8
_NST = _RPW // _CH


_NPAD = B + 8


def _sc_gather_body(table_hbm, qt_hbm, outq_hbm, outs_hbm,
                    idx_e, idx_o, idxs_v, out_v, outs_v, sem, sem_s):
    wid = lax.axis_index("s") * 2 + lax.axis_index("c")
    base = wid * _RPW
    pltpu.sync_copy(qt_hbm.at[pl.ds(base, _RPW)], idx_e)
    pltpu.sync_copy(qt_hbm.at[pl.ds(_NPAD + base, _RPW)], idx_o)
    copies = [
        pltpu.async_copy(
            table_hbm.at[idx_e.at[pl.ds(j * _CH, _CH)]],
            out_v.at[pl.ds(j * _CH, _CH), pl.ds(0, D)], sem)
        for j in range(_NST)
    ] + [
        pltpu.async_copy(
            table_hbm.at[idx_o.at[pl.ds(j * _CH, _CH)]],
            out_v.at[pl.ds(j * _CH, _CH), pl.ds(D, D)], sem)
        for j in range(_NST)
    ]

    @pl.when(wid == 0)
    def _():
        pltpu.sync_copy(qt_hbm.at[pl.ds(B, 8)], idxs_v.at[pl.ds(0, 8)])
        pltpu.sync_copy(qt_hbm.at[pl.ds(_NPAD + B, 8)],
                        idxs_v.at[pl.ds(8, 8)])
        pltpu.async_copy(table_hbm.at[idxs_v.at[pl.ds(0, 8)]],
                         outs_v.at[:, pl.ds(0, D)], sem_s)
        pltpu.async_copy(table_hbm.at[idxs_v.at[pl.ds(8, 8)]],
                         outs_v.at[:, pl.ds(D, D)], sem_s)

    for c in copies:
        c.wait()
    pltpu.sync_copy(out_v, outq_hbm.at[pl.ds(base, _RPW)])

    @pl.when(wid == 0)
    def _():
        pltpu.make_async_copy(table_hbm.at[idxs_v.at[pl.ds(0, 8)]],
                              outs_v.at[:, pl.ds(0, D)], sem_s).wait()
        pltpu.make_async_copy(table_hbm.at[idxs_v.at[pl.ds(8, 8)]],
                              outs_v.at[:, pl.ds(D, D)], sem_s).wait()
        pltpu.sync_copy(outs_v, outs_hbm)


@functools.cache
def _make_sc_gather():
    return pl.kernel(
        _sc_gather_body,
        out_type=(
            jax.ShapeDtypeStruct((B, D2), jnp.float32),
            jax.ShapeDtypeStruct((8, D2), jnp.float32),
        ),
        mesh=plsc.VectorSubcoreMesh(core_axis_name="c", subcore_axis_name="s"),
        scratch_types=[
            pltpu.VMEM((_RPW,), jnp.int32),
            pltpu.VMEM((_RPW,), jnp.int32),
            pltpu.VMEM((16,), jnp.int32),
            pltpu.VMEM((_RPW, D2), jnp.float32),
            pltpu.VMEM((8, D2), jnp.float32),
            pltpu.SemaphoreType.DMA,
            pltpu.SemaphoreType.DMA,
        ],
    )


def _sc_gather(table, qt_flat):
    return _make_sc_gather()(table, qt_flat)


def _sigmoid(x):
    return 0.5 * jnp.tanh(0.5 * x) + 0.5


def _encode(x, W1, b1, W2, b2, ln_g, ln_b):
    h = jnp.maximum(jnp.dot(x, W1, preferred_element_type=jnp.float32) + b1, 0.0)
    h = jnp.dot(h, W2, preferred_element_type=jnp.float32) + b2
    y = h + x
    mu = jnp.mean(y, axis=-1, keepdims=True)
    var = jnp.mean((y - mu) * (y - mu), axis=-1, keepdims=True)
    return ln_g * (y - mu) * lax.rsqrt(var + 1e-5) + ln_b


def _dot_nt(x, w):
    return lax.dot_general(x, w, (((1,), (1,)), ((), ())),
                           preferred_element_type=jnp.float32)


_G4 = 4 * D2


_NBT = 4
_BT = B // _NBT


def _weight_copies(Wih_hbm, Whh_hbm, wih_v, whh_v, sem_ih, sem_hh):
    return (
        [pltpu.make_async_copy(Wih_hbm.at[pl.ds(k * HID, D2)],
                               wih_v.at[pl.ds(k * D2, D2)], sem_ih)
         for k in range(4)],
        [pltpu.make_async_copy(Whh_hbm.at[pl.ds(k * HID, D2)],
                               whh_v.at[pl.ds(k * D2, D2)], sem_hh)
         for k in range(4)],
    )


def _tc_body(q_ref, s_ref, W1_ref, b1_ref, W2_ref, b2_ref, lng_ref, lnb_ref,
             Wih_hbm, Whh_hbm, bih_ref, bhh_ref, out_ref,
             wih_v, whh_v, smean_v, rrow_v, sem_ih, sem_hh):
    pid = pl.program_id(0)
    cps_ih, cps_hh = _weight_copies(
        Wih_hbm, Whh_hbm, wih_v, whh_v, sem_ih, sem_hh)

    W1 = W1_ref[...]
    b1 = b1_ref[...]
    W2 = W2_ref[...]
    b2 = b2_ref[...]
    ln_g = lng_ref[...]
    ln_b = lnb_ref[...]

    @pl.when(pid == 0)
    def _():
        for cp in cps_ih:
            cp.start()
        for cp in cps_hh:
            cp.start()
        s_g = _encode(s_ref[...], W1, b1, W2, b2, ln_g, ln_b)
        row = lax.broadcasted_iota(jnp.int32, (8, 1), 0)
        s_g = jnp.where(row < FEW, s_g, 0.0)
        smean_v[...] = jnp.sum(s_g, axis=0, keepdims=True) * (1.0 / FEW)

    q_g = _encode(q_ref[...], W1, b1, W2, b2, ln_g, ln_b)

    bsum = bih_ref[...] + bhh_ref[...]
    b4 = jnp.concatenate(
        [bsum[:, k * HID:k * HID + D2] for k in range(4)], axis=1)

    @pl.when(pid == 0)
    def _():
        for cp in cps_ih:
            cp.wait()

    a = _dot_nt(q_g, wih_v[...]) + b4

    @pl.when(pid == 0)
    def _():
        for cp in cps_hh:
            cp.wait()
        rrow_v[...] = _dot_nt(smean_v[...], whh_v[:, D2:])

    s_mean = smean_v[...]
    r_row = rrow_v[...]
    Whh_h = whh_v[:, :D2]

    c = None
    h = None
    gates = a
    for step in range(STEPS):
        if step > 0:
            gates = a + r_row + _dot_nt(h, Whh_h)
        i = _sigmoid(gates[:, :D2])
        f = _sigmoid(gates[:, D2:2 * D2])
        g = jnp.tanh(gates[:, 2 * D2:3 * D2])
        o = _sigmoid(gates[:, 3 * D2:])
        c = f * c + i * g if step > 0 else i * g
        h = q_g + o * jnp.tanh(c)

    out_ref[...] = jnp.sum(h * s_mean, axis=1, keepdims=True)


@jax.jit
def _tc_dense(q, s, W1, b1, W2, b2, ln_g, ln_b, W_ih, W_hh, b_ih, b_hh):
    full = lambda shape: pl.BlockSpec(shape, lambda *_: (0,) * len(shape))
    hbm = pl.BlockSpec(memory_space=pl.ANY)
    return pl.pallas_call(
        _tc_body,
        grid=(_NBT,),
        in_specs=[
            pl.BlockSpec((_BT, D2), lambda i: (i, 0)),
            full((8, D2)),
            full((D2, 2 * D2)),
            full((1, 2 * D2)),
            full((2 * D2, D2)),
            full((1, D2)),
            full((1, D2)),
            full((1, D2)),
            hbm,
            hbm,
            full((1, H4)),
            full((1, H4)),
        ],
        out_specs=pl.BlockSpec((_BT, 1), lambda i: (i, 0)),
        out_shape=jax.ShapeDtypeStruct((B, 1), jnp.float32),
        scratch_shapes=[
            pltpu.VMEM((_G4, D2), jnp.float32),
            pltpu.VMEM((_G4, HID), jnp.float32),
            pltpu.VMEM((1, D2), jnp.float32),
            pltpu.VMEM((1, _G4), jnp.float32),
            pltpu.SemaphoreType.DMA,
            pltpu.SemaphoreType.DMA,
        ],
    )(q, s, W1, b1, W2, b2, ln_g, ln_b, W_ih, W_hh, b_ih, b_hh)


def kernel(query, support, symbol_emb, W1, b1, W2, b2, ln_g, ln_b, W_ih, W_hh, b_ih, b_hh):
    if query.dtype != jnp.int32:
        query = query.astype(jnp.int32)
    if support.dtype != jnp.int32:
        support = support.astype(jnp.int32)
    qs = jnp.concatenate([query, support, jnp.zeros((3, 2), jnp.int32)])
    q, s = _sc_gather(symbol_emb, qs.T.reshape(-1))

    scores = _tc_dense(
        q, s, W1, b1.reshape(1, -1), W2, b2.reshape(1, -1),
        ln_g.reshape(1, -1), ln_b.reshape(1, -1),
        W_ih, W_hh, b_ih.reshape(1, -1), b_hh.reshape(1, -1))
    return scores.reshape(B)


# speedup vs baseline: 1.0325x; 1.0325x over previous
import functools

import jax
import jax.numpy as jnp
from jax import lax
from jax.experimental import pallas as pl
from jax.experimental.pallas import tpu as pltpu
from jax.experimental.pallas import tpu_sc as plsc

D = 128
D2 = 2 * D
HID = 2 * D2
H4 = 4 * HID
B = 1024
FEW = 5
STEPS = 4


_NW = 32
_RPW = B // _NW
_CH = 8
_NST = _RPW // _CH


_NPAD = B + 8


def _sc_gather_body(table_hbm, qt_hbm, outq_hbm, outs_hbm,
                    idx_e, idx_o, idxs_v, out_v, outs_v, sem, sem_s):
    wid = lax.axis_index("s") * 2 + lax.axis_index("c")
    base = wid * _RPW
    pltpu.sync_copy(qt_hbm.at[pl.ds(base, _RPW)], idx_e)
    pltpu.sync_copy(qt_hbm.at[pl.ds(_NPAD + base, _RPW)], idx_o)
    copies = [
        pltpu.async_copy(
            table_hbm.at[idx_e.at[pl.ds(j * _CH, _CH)]],
            out_v.at[pl.ds(j * _CH, _CH), pl.ds(0, D)], sem)
        for j in range(_NST)
    ] + [
        pltpu.async_copy(
            table_hbm.at[idx_o.at[pl.ds(j * _CH, _CH)]],
            out_v.at[pl.ds(j * _CH, _CH), pl.ds(D, D)], sem)
        for j in range(_NST)
    ]

    @pl.when(wid == 0)
    def _():
        pltpu.sync_copy(qt_hbm.at[pl.ds(B, 8)], idxs_v.at[pl.ds(0, 8)])
        pltpu.sync_copy(qt_hbm.at[pl.ds(_NPAD + B, 8)],
                        idxs_v.at[pl.ds(8, 8)])
        pltpu.async_copy(table_hbm.at[idxs_v.at[pl.ds(0, 8)]],
                         outs_v.at[:, pl.ds(0, D)], sem_s)
        pltpu.async_copy(table_hbm.at[idxs_v.at[pl.ds(8, 8)]],
                         outs_v.at[:, pl.ds(D, D)], sem_s)

    for c in copies:
        c.wait()
    pltpu.sync_copy(out_v, outq_hbm.at[pl.ds(base, _RPW)])

    @pl.when(wid == 0)
    def _():
        pltpu.make_async_copy(table_hbm.at[idxs_v.at[pl.ds(0, 8)]],
                              outs_v.at[:, pl.ds(0, D)], sem_s).wait()
        pltpu.make_async_copy(table_hbm.at[idxs_v.at[pl.ds(8, 8)]],
                              outs_v.at[:, pl.ds(D, D)], sem_s).wait()
        pltpu.sync_copy(outs_v, outs_hbm)


@functools.cache
def _make_sc_gather():
    return pl.kernel(
        _sc_gather_body,
        out_type=(
            jax.ShapeDtypeStruct((B, D2), jnp.float32),
            jax.ShapeDtypeStruct((8, D2), jnp.float32),
        ),
        mesh=plsc.VectorSubcoreMesh(core_axis_name="c", subcore_axis_name="s"),
        scratch_types=[
            pltpu.VMEM((_RPW,), jnp.int32),
            pltpu.VMEM((_RPW,), jnp.int32),
            pltpu.VMEM((16,), jnp.int32),
            pltpu.VMEM((_RPW, D2), jnp.float32),
            pltpu.VMEM((8, D2), jnp.float32),
            pltpu.SemaphoreType.DMA,
            pltpu.SemaphoreType.DMA,
        ],
    )


def _sc_gather(table, qt_flat):
    return _make_sc_gather()(table, qt_flat)


def _sigmoid(x):
    return 0.5 * jnp.tanh(0.5 * x) + 0.5


def _encode(x, W1, b1, W2, b2, ln_g, ln_b):
    h = jnp.maximum(jnp.dot(x, W1, preferred_element_type=jnp.float32) + b1, 0.0)
    h = jnp.dot(h, W2, preferred_element_type=jnp.float32) + b2
    y = h + x
    mu = jnp.mean(y, axis=-1, keepdims=True)
    var = jnp.mean((y - mu) * (y - mu), axis=-1, keepdims=True)
    return ln_g * (y - mu) * lax.rsqrt(var + 1e-5) + ln_b


def _dot_nt(x, w):
    return lax.dot_general(x, w, (((1,), (1,)), ((), ())),
                           preferred_element_type=jnp.float32)


_G4 = 4 * D2


_NBT = 4
_BT = B // _NBT


def _weight_copies(Wih_hbm, Whh_hbm, wih_v, whh_v, sem_ih, sem_hh):
    return (
        [pltpu.make_async_copy(Wih_hbm.at[pl.ds(k * HID, D2)],
                               wih_v.at[pl.ds(k * D2, D2)], sem_ih)
         for k in range(4)],
        [pltpu.make_async_copy(Whh_hbm.at[pl.ds(k * HID, D2)],
                               whh_v.at[pl.ds(k * D2, D2)], sem_hh)
         for k in range(4)],
    )


def _tc_body(q_ref, s_ref, W1_ref, b1_ref, W2_ref, b2_ref, lng_ref, lnb_ref,
             Wih_hbm, Whh_hbm, bih_ref, bhh_ref, out_ref,
             wih_v, whh_v, smean_v, rrow_v, sem_ih, sem_hh):
    pid = pl.program_id(0)
    cps_ih, cps_hh = _weight_copies(
        Wih_hbm, Whh_hbm, wih_v, whh_v, sem_ih, sem_hh)

    W1 = W1_ref[...]
    b1 = b1_ref[...]
    W2 = W2_ref[...]
    b2 = b2_ref[...]
    ln_g = lng_ref[...]
    ln_b = lnb_ref[...]

    @pl.when(pid == 0)
    def _():
        for cp in cps_ih:
            cp.start()
        for cp in cps_hh:
            cp.start()
        s_g = _encode(s_ref[...], W1, b1, W2, b2, ln_g, ln_b)
        row = lax.broadcasted_iota(jnp.int32, (8, 1), 0)
        s_g = jnp.where(row < FEW, s_g, 0.0)
        smean_v[...] = jnp.sum(s_g, axis=0, keepdims=True) * (1.0 / FEW)

    q_g = _encode(q_ref[...], W1, b1, W2, b2, ln_g, ln_b)

    bsum = bih_ref[...] + bhh_ref[...]
    b4 = jnp.concatenate(
        [bsum[:, k * HID:k * HID + D2] for k in range(4)], axis=1)

    @pl.when(pid == 0)
    def _():
        for cp in cps_ih:
            cp.wait()
        for cp in cps_hh:
            cp.wait()
        rrow_v[...] = _dot_nt(smean_v[...], whh_v[:, D2:])

    s_mean = smean_v[...]
    r_row = rrow_v[...]
    Whh_h = whh_v[:, :D2]
    a = _dot_nt(q_g, wih_v[...]) + b4

    c = None
    h = None
    gates = a
    for step in range(STEPS):
        if step > 0:
            gates = a + r_row + _dot_nt(h, Whh_h)
        i = _sigmoid(gates[:, :D2])
        f = _sigmoid(gates[:, D2:2 * D2])
        g = jnp.tanh(gates[:, 2 * D2:3 * D2])
        o = _sigmoid(gates[:, 3 * D2:])
        c = f * c + i * g if step > 0 else i * g
        h = q_g + o * jnp.tanh(c)

    out_ref[...] = jnp.sum(h * s_mean, axis=1, keepdims=True)


@jax.jit
def _tc_dense(q, s, W1, b1, W2, b2, ln_g, ln_b, W_ih, W_hh, b_ih, b_hh):
    full = lambda shape: pl.BlockSpec(shape, lambda *_: (0,) * len(shape))
    hbm = pl.BlockSpec(memory_space=pl.ANY)
    return pl.pallas_call(
        _tc_body,
        grid=(_NBT,),
        in_specs=[
            pl.BlockSpec((_BT, D2), lambda i: (i, 0)),
            full((8, D2)),
            full((D2, 2 * D2)),
            full((1, 2 * D2)),
            full((2 * D2, D2)),
            full((1, D2)),
            full((1, D2)),
            full((1, D2)),
            hbm,
            hbm,
            full((1, H4)),
            full((1, H4)),
        ],
        out_specs=pl.BlockSpec((_BT, 1), lambda i: (i, 0)),
        out_shape=jax.ShapeDtypeStruct((B, 1), jnp.float32),
        scratch_shapes=[
            pltpu.VMEM((_G4, D2), jnp.float32),
            pltpu.VMEM((_G4, HID), jnp.float32),
            pltpu.VMEM((1, D2), jnp.float32),
            pltpu.VMEM((1, _G4), jnp.float32),
            pltpu.SemaphoreType.DMA,
            pltpu.SemaphoreType.DMA,
        ],
    )(q, s, W1, b1, W2, b2, ln_g, ln_b, W_ih, W_hh, b_ih, b_hh)


def kernel(query, support, symbol_emb, W1, b1, W2, b2, ln_g, ln_b, W_ih, W_hh, b_ih, b_hh):
    if query.dtype != jnp.int32:
        query = query.astype(jnp.int32)
    if support.dtype != jnp.int32:
        support = support.astype(jnp.int32)
    qs = jnp.concatenate([query, support, jnp.zeros((3, 2), jnp.int32)])
    q, s = _sc_gather(symbol_emb, qs.T.reshape(-1))

    scores = _tc_dense(
        q, s, W1, b1.reshape(1, -1), W2, b2.reshape(1, -1),
        ln_g.reshape(1, -1), ln_b.reshape(1, -1),
        W_ih, W_hh, b_ih.reshape(1, -1), b_hh.reshape(1, -1))
    return scores.reshape(B)


# speedup vs baseline: 1.1149x; 1.0799x over previous
import functools

import jax
import jax.numpy as jnp
from jax import lax
from jax.experimental import pallas as pl
from jax.experimental.pallas import tpu as pltpu
from jax.experimental.pallas import tpu_sc as plsc

D = 128
D2 = 2 * D
HID = 2 * D2
H4 = 4 * HID
B = 1024
FEW = 5
STEPS = 4


_NW = 32
_RPW = B // _NW
_CH = 8
_NST = _RPW // _CH


_NPAD = B + 8


def _sc_gather_body(table_hbm, qt_hbm, outq_hbm, outs_hbm,
                    idx_e, idx_o, idxs_v, out_v, outs_v, sem, sem_s):
    wid = lax.axis_index("s") * 2 + lax.axis_index("c")
    base = wid * _RPW
    pltpu.sync_copy(qt_hbm.at[pl.ds(base, _RPW)], idx_e)
    pltpu.sync_copy(qt_hbm.at[pl.ds(_NPAD + base, _RPW)], idx_o)
    copies = [
        pltpu.async_copy(
            table_hbm.at[idx_e.at[pl.ds(j * _CH, _CH)]],
            out_v.at[pl.ds(j * _CH, _CH), pl.ds(0, D)], sem)
        for j in range(_NST)
    ] + [
        pltpu.async_copy(
            table_hbm.at[idx_o.at[pl.ds(j * _CH, _CH)]],
            out_v.at[pl.ds(j * _CH, _CH), pl.ds(D, D)], sem)
        for j in range(_NST)
    ]

    @pl.when(wid == 0)
    def _():
        pltpu.sync_copy(qt_hbm.at[pl.ds(B, 8)], idxs_v.at[pl.ds(0, 8)])
        pltpu.sync_copy(qt_hbm.at[pl.ds(_NPAD + B, 8)],
                        idxs_v.at[pl.ds(8, 8)])
        pltpu.async_copy(table_hbm.at[idxs_v.at[pl.ds(0, 8)]],
                         outs_v.at[:, pl.ds(0, D)], sem_s)
        pltpu.async_copy(table_hbm.at[idxs_v.at[pl.ds(8, 8)]],
                         outs_v.at[:, pl.ds(D, D)], sem_s)

    for c in copies:
        c.wait()
    pltpu.sync_copy(out_v, outq_hbm.at[pl.ds(base, _RPW)])

    @pl.when(wid == 0)
    def _():
        pltpu.make_async_copy(table_hbm.at[idxs_v.at[pl.ds(0, 8)]],
                              outs_v.at[:, pl.ds(0, D)], sem_s).wait()
        pltpu.make_async_copy(table_hbm.at[idxs_v.at[pl.ds(8, 8)]],
                              outs_v.at[:, pl.ds(D, D)], sem_s).wait()
        pltpu.sync_copy(outs_v, outs_hbm)


@functools.cache
def _make_sc_gather():
    return pl.kernel(
        _sc_gather_body,
        out_type=(
            jax.ShapeDtypeStruct((B, D2), jnp.float32),
            jax.ShapeDtypeStruct((8, D2), jnp.float32),
        ),
        mesh=plsc.VectorSubcoreMesh(core_axis_name="c", subcore_axis_name="s"),
        scratch_types=[
            pltpu.VMEM((_RPW,), jnp.int32),
            pltpu.VMEM((_RPW,), jnp.int32),
            pltpu.VMEM((16,), jnp.int32),
            pltpu.VMEM((_RPW, D2), jnp.float32),
            pltpu.VMEM((8, D2), jnp.float32),
            pltpu.SemaphoreType.DMA,
            pltpu.SemaphoreType.DMA,
        ],
    )


def _sc_gather(table, qt_flat):
    return _make_sc_gather()(table, qt_flat)


def _sigmoid(x):
    return 0.5 * jnp.tanh(0.5 * x) + 0.5


def _encode(x, W1, b1, W2, b2, ln_g, ln_b):
    h = jnp.maximum(jnp.dot(x, W1, preferred_element_type=jnp.float32) + b1, 0.0)
    h = jnp.dot(h, W2, preferred_element_type=jnp.float32) + b2
    y = h + x
    mu = jnp.mean(y, axis=-1, keepdims=True)
    var = jnp.mean((y - mu) * (y - mu), axis=-1, keepdims=True)
    return ln_g * (y - mu) * lax.rsqrt(var + 1e-5) + ln_b


def _dot_nt(x, w):
    return lax.dot_general(x, w, (((1,), (1,)), ((), ())),
                           preferred_element_type=jnp.float32)


_G4 = 4 * D2


_NBT = 2
_BT = B // _NBT


def _weight_copies(Wih_hbm, Whh_hbm, wih_v, whh_v, sem_ih, sem_hh):
    return (
        [pltpu.make_async_copy(Wih_hbm.at[pl.ds(k * HID, D2)],
                               wih_v.at[pl.ds(k * D2, D2)], sem_ih)
         for k in range(4)],
        [pltpu.make_async_copy(Whh_hbm.at[pl.ds(k * HID, D2)],
                               whh_v.at[pl.ds(k * D2, D2)], sem_hh)
         for k in range(4)],
    )


def _tc_body(q_ref, s_ref, W1_ref, b1_ref, W2_ref, b2_ref, lng_ref, lnb_ref,
             Wih_hbm, Whh_hbm, bih_ref, bhh_ref, out_ref,
             wih_v, whh_v, smean_v, rrow_v, sem_ih, sem_hh):
    pid = pl.program_id(0)
    cps_ih, cps_hh = _weight_copies(
        Wih_hbm, Whh_hbm, wih_v, whh_v, sem_ih, sem_hh)

    W1 = W1_ref[...]
    b1 = b1_ref[...]
    W2 = W2_ref[...]
    b2 = b2_ref[...]
    ln_g = lng_ref[...]
    ln_b = lnb_ref[...]

    @pl.when(pid == 0)
    def _():
        for cp in cps_ih:
            cp.start()
        for cp in cps_hh:
            cp.start()
        s_g = _encode(s_ref[...], W1, b1, W2, b2, ln_g, ln_b)
        row = lax.broadcasted_iota(jnp.int32, (8, 1), 0)
        s_g = jnp.where(row < FEW, s_g, 0.0)
        smean_v[...] = jnp.sum(s_g, axis=0, keepdims=True) * (1.0 / FEW)

    q_g = _encode(q_ref[...], W1, b1, W2, b2, ln_g, ln_b)

    bsum = bih_ref[...] + bhh_ref[...]
    b4 = jnp.concatenate(
        [bsum[:, k * HID:k * HID + D2] for k in range(4)], axis=1)

    @pl.when(pid == 0)
    def _():
        for cp in cps_ih:
            cp.wait()
        for cp in cps_hh:
            cp.wait()
        rrow_v[...] = _dot_nt(smean_v[...], whh_v[:, D2:])

    s_mean = smean_v[...]
    r_row = rrow_v[...]
    Whh_h = whh_v[:, :D2]
    a = _dot_nt(q_g, wih_v[...]) + b4

    c = None
    h = None
    gates = a
    for step in range(STEPS):
        if step > 0:
            gates = a + r_row + _dot_nt(h, Whh_h)
        i = _sigmoid(gates[:, :D2])
        f = _sigmoid(gates[:, D2:2 * D2])
        g = jnp.tanh(gates[:, 2 * D2:3 * D2])
        o = _sigmoid(gates[:, 3 * D2:])
        c = f * c + i * g if step > 0 else i * g
        h = q_g + o * jnp.tanh(c)

    out_ref[...] = jnp.sum(h * s_mean, axis=1, keepdims=True)


@jax.jit
def _tc_dense(q, s, W1, b1, W2, b2, ln_g, ln_b, W_ih, W_hh, b_ih, b_hh):
    full = lambda shape: pl.BlockSpec(shape, lambda *_: (0,) * len(shape))
    hbm = pl.BlockSpec(memory_space=pl.ANY)
    return pl.pallas_call(
        _tc_body,
        grid=(_NBT,),
        in_specs=[
            pl.BlockSpec((_BT, D2), lambda i: (i, 0)),
            full((8, D2)),
            full((D2, 2 * D2)),
            full((1, 2 * D2)),
            full((2 * D2, D2)),
            full((1, D2)),
            full((1, D2)),
            full((1, D2)),
            hbm,
            hbm,
            full((1, H4)),
            full((1, H4)),
        ],
        out_specs=pl.BlockSpec((_BT, 1), lambda i: (i, 0)),
        out_shape=jax.ShapeDtypeStruct((B, 1), jnp.float32),
        scratch_shapes=[
            pltpu.VMEM((_G4, D2), jnp.float32),
            pltpu.VMEM((_G4, HID), jnp.float32),
            pltpu.VMEM((1, D2), jnp.float32),
            pltpu.VMEM((1, _G4), jnp.float32),
            pltpu.SemaphoreType.DMA,
            pltpu.SemaphoreType.DMA,
        ],
    )(q, s, W1, b1, W2, b2, ln_g, ln_b, W_ih, W_hh, b_ih, b_hh)


def kernel(query, support, symbol_emb, W1, b1, W2, b2, ln_g, ln_b, W_ih, W_hh, b_ih, b_hh):
    if query.dtype != jnp.int32:
        query = query.astype(jnp.int32)
    if support.dtype != jnp.int32:
        support = support.astype(jnp.int32)
    qs = jnp.concatenate([query, support, jnp.zeros((3, 2), jnp.int32)])
    q, s = _sc_gather(symbol_emb, qs.T.reshape(-1))

    scores = _tc_dense(
        q, s, W1, b1.reshape(1, -1), W2, b2.reshape(1, -1),
        ln_g.reshape(1, -1), ln_b.reshape(1, -1),
        W_ih, W_hh, b_ih.reshape(1, -1), b_hh.reshape(1, -1))
    return scores.reshape(B)


# speedup vs baseline: 1.1483x; 1.0299x over previous
import functools

import jax
import jax.numpy as jnp
from jax import lax
from jax.experimental import pallas as pl
from jax.experimental.pallas import tpu as pltpu
from jax.experimental.pallas import tpu_sc as plsc

D = 128
D2 = 2 * D
HID = 2 * D2
H4 = 4 * HID
B = 1024
FEW = 5
STEPS = 4


_NW = 32
_RPW = B // _NW
_CH = 8
_NST = _RPW // _CH


_NPAD = B + 8


def _sc_gather_body(table_hbm, qt_hbm, outq_hbm, outs_hbm,
                    idx_e, idx_o, idxs_v, out_v, outs_v, sem, sem_s):
    wid = lax.axis_index("s") * 2 + lax.axis_index("c")
    base = wid * _RPW
    pltpu.sync_copy(qt_hbm.at[pl.ds(base, _RPW)], idx_e)
    pltpu.sync_copy(qt_hbm.at[pl.ds(_NPAD + base, _RPW)], idx_o)
    copies = [
        pltpu.async_copy(
            table_hbm.at[idx_e.at[pl.ds(j * _CH, _CH)]],
            out_v.at[pl.ds(j * _CH, _CH), pl.ds(0, D)], sem)
        for j in range(_NST)
    ] + [
        pltpu.async_copy(
            table_hbm.at[idx_o.at[pl.ds(j * _CH, _CH)]],
            out_v.at[pl.ds(j * _CH, _CH), pl.ds(D, D)], sem)
        for j in range(_NST)
    ]

    @pl.when(wid == 0)
    def _():
        pltpu.sync_copy(qt_hbm.at[pl.ds(B, 8)], idxs_v.at[pl.ds(0, 8)])
        pltpu.sync_copy(qt_hbm.at[pl.ds(_NPAD + B, 8)],
                        idxs_v.at[pl.ds(8, 8)])
        pltpu.async_copy(table_hbm.at[idxs_v.at[pl.ds(0, 8)]],
                         outs_v.at[:, pl.ds(0, D)], sem_s)
        pltpu.async_copy(table_hbm.at[idxs_v.at[pl.ds(8, 8)]],
                         outs_v.at[:, pl.ds(D, D)], sem_s)

    for c in copies:
        c.wait()
    pltpu.sync_copy(out_v, outq_hbm.at[pl.ds(base, _RPW)])

    @pl.when(wid == 0)
    def _():
        pltpu.make_async_copy(table_hbm.at[idxs_v.at[pl.ds(0, 8)]],
                              outs_v.at[:, pl.ds(0, D)], sem_s).wait()
        pltpu.make_async_copy(table_hbm.at[idxs_v.at[pl.ds(8, 8)]],
                              outs_v.at[:, pl.ds(D, D)], sem_s).wait()
        pltpu.sync_copy(outs_v, outs_hbm)


@functools.cache
def _make_sc_gather():
    return pl.kernel(
        _sc_gather_body,
        out_type=(
            jax.ShapeDtypeStruct((B, D2), jnp.float32),
            jax.ShapeDtypeStruct((8, D2), jnp.float32),
        ),
        mesh=plsc.VectorSubcoreMesh(core_axis_name="c", subcore_axis_name="s"),
        scratch_types=[
            pltpu.VMEM((_RPW,), jnp.int32),
            pltpu.VMEM((_RPW,), jnp.int32),
            pltpu.VMEM((16,), jnp.int32),
            pltpu.VMEM((_RPW, D2), jnp.float32),
            pltpu.VMEM((8, D2), jnp.float32),
            pltpu.SemaphoreType.DMA,
            pltpu.SemaphoreType.DMA,
        ],
    )


def _sc_gather(table, qt_flat):
    return _make_sc_gather()(table, qt_flat)


def _sigmoid(x):
    return 0.5 * jnp.tanh(0.5 * x) + 0.5


def _encode(x, W1, b1, W2, b2, ln_g, ln_b):
    h = jnp.maximum(jnp.dot(x, W1, preferred_element_type=jnp.float32) + b1, 0.0)
    h = jnp.dot(h, W2, preferred_element_type=jnp.float32) + b2
    y = h + x
    mu = jnp.mean(y, axis=-1, keepdims=True)
    var = jnp.mean((y - mu) * (y - mu), axis=-1, keepdims=True)
    return ln_g * (y - mu) * lax.rsqrt(var + 1e-5) + ln_b


def _dot_nt(x, w):
    return lax.dot_general(x, w, (((1,), (1,)), ((), ())),
                           preferred_element_type=jnp.float32)


_G4 = 4 * D2


_NBT = 1
_BT = B // _NBT


def _weight_copies(Wih_hbm, Whh_hbm, wih_v, whh_v, sem_ih, sem_hh):
    return (
        [pltpu.make_async_copy(Wih_hbm.at[pl.ds(k * HID, D2)],
                               wih_v.at[pl.ds(k * D2, D2)], sem_ih)
         for k in range(4)],
        [pltpu.make_async_copy(Whh_hbm.at[pl.ds(k * HID, D2)],
                               whh_v.at[pl.ds(k * D2, D2)], sem_hh)
         for k in range(4)],
    )


def _tc_body(q_ref, s_ref, W1_ref, b1_ref, W2_ref, b2_ref, lng_ref, lnb_ref,
             Wih_hbm, Whh_hbm, bih_ref, bhh_ref, out_ref,
             wih_v, whh_v, smean_v, rrow_v, sem_ih, sem_hh):
    pid = pl.program_id(0)
    cps_ih, cps_hh = _weight_copies(
        Wih_hbm, Whh_hbm, wih_v, whh_v, sem_ih, sem_hh)

    W1 = W1_ref[...]
    b1 = b1_ref[...]
    W2 = W2_ref[...]
    b2 = b2_ref[...]
    ln_g = lng_ref[...]
    ln_b = lnb_ref[...]

    @pl.when(pid == 0)
    def _():
        for cp in cps_ih:
            cp.start()
        for cp in cps_hh:
            cp.start()
        s_g = _encode(s_ref[...], W1, b1, W2, b2, ln_g, ln_b)
        row = lax.broadcasted_iota(jnp.int32, (8, 1), 0)
        s_g = jnp.where(row < FEW, s_g, 0.0)
        smean_v[...] = jnp.sum(s_g, axis=0, keepdims=True) * (1.0 / FEW)

    q_g = _encode(q_ref[...], W1, b1, W2, b2, ln_g, ln_b)

    bsum = bih_ref[...] + bhh_ref[...]
    b4 = jnp.concatenate(
        [bsum[:, k * HID:k * HID + D2] for k in range(4)], axis=1)

    @pl.when(pid == 0)
    def _():
        for cp in cps_ih:
            cp.wait()
        for cp in cps_hh:
            cp.wait()
        rrow_v[...] = _dot_nt(smean_v[...], whh_v[:, D2:])

    s_mean = smean_v[...]
    r_row = rrow_v[...]
    Whh_h = whh_v[:, :D2]
    a = _dot_nt(q_g, wih_v[...]) + b4

    c = None
    h = None
    gates = a
    for step in range(STEPS):
        if step > 0:
            gates = a + r_row + _dot_nt(h, Whh_h)
        i = _sigmoid(gates[:, :D2])
        f = _sigmoid(gates[:, D2:2 * D2])
        g = jnp.tanh(gates[:, 2 * D2:3 * D2])
        o = _sigmoid(gates[:, 3 * D2:])
        c = f * c + i * g if step > 0 else i * g
        h = q_g + o * jnp.tanh(c)

    out_ref[...] = jnp.sum(h * s_mean, axis=1, keepdims=True)


@jax.jit
def _tc_dense(q, s, W1, b1, W2, b2, ln_g, ln_b, W_ih, W_hh, b_ih, b_hh):
    full = lambda shape: pl.BlockSpec(shape, lambda *_: (0,) * len(shape))
    hbm = pl.BlockSpec(memory_space=pl.ANY)
    return pl.pallas_call(
        _tc_body,
        grid=(_NBT,),
        in_specs=[
            pl.BlockSpec((_BT, D2), lambda i: (i, 0)),
            full((8, D2)),
            full((D2, 2 * D2)),
            full((1, 2 * D2)),
            full((2 * D2, D2)),
            full((1, D2)),
            full((1, D2)),
            full((1, D2)),
            hbm,
            hbm,
            full((1, H4)),
            full((1, H4)),
        ],
        out_specs=pl.BlockSpec((_BT, 1), lambda i: (i, 0)),
        out_shape=jax.ShapeDtypeStruct((B, 1), jnp.float32),
        scratch_shapes=[
            pltpu.VMEM((_G4, D2), jnp.float32),
            pltpu.VMEM((_G4, HID), jnp.float32),
            pltpu.VMEM((1, D2), jnp.float32),
            pltpu.VMEM((1, _G4), jnp.float32),
            pltpu.SemaphoreType.DMA,
            pltpu.SemaphoreType.DMA,
        ],
    )(q, s, W1, b1, W2, b2, ln_g, ln_b, W_ih, W_hh, b_ih, b_hh)


def kernel(query, support, symbol_emb, W1, b1, W2, b2, ln_g, ln_b, W_ih, W_hh, b_ih, b_hh):
    if query.dtype != jnp.int32:
        query = query.astype(jnp.int32)
    if support.dtype != jnp.int32:
        support = support.astype(jnp.int32)
    qs = jnp.concatenate([query, support, jnp.zeros((3, 2), jnp.int32)])
    q, s = _sc_gather(symbol_emb, qs.T.reshape(-1))

    scores = _tc_dense(
        q, s, W1, b1.reshape(1, -1), W2, b2.reshape(1, -1),
        ln_g.reshape(1, -1), ln_b.reshape(1, -1),
        W_ih, W_hh, b_ih.reshape(1, -1), b_hh.reshape(1, -1))
    return scores.reshape(B)
